# Initial kernel scaffold; baseline (speedup 1.0000x reference)
import jax
import jax.numpy as jnp
from jax.experimental import pallas as pl


def kernel(x, edge_index, W1, b1, W2, b2):
    raise NotImplementedError("write your pallas kernel here")


# speedup vs baseline: 5.5692x; 5.5692x over previous
import functools

import jax
import jax.numpy as jnp
from jax import lax
from jax.experimental import pallas as pl
from jax.experimental.pallas import tpu as pltpu
from jax.experimental.pallas import tpu_sc as plsc

N = 10000
E = 320000
NPAD = 10240
F = 128
C = 40
NC = 2
NS = 16
NW = NC * NS
EPW = E // NW
CH = 80
NCHUNK = EPW // CH
RPT = NPAD // NS
BR = 1024
GR = NPAD // BR

_MESH = plsc.VectorSubcoreMesh(
    core_axis_name="c", subcore_axis_name="s", num_cores=NC, num_subcores=NS)


@functools.partial(
    pl.kernel,
    out_type=(jax.ShapeDtypeStruct((NC, NPAD), jnp.float32),
              jax.ShapeDtypeStruct((NC, NPAD), jnp.float32)),
    mesh=_MESH,
    scratch_types=(
        pltpu.VMEM((CH,), jnp.int32),
        pltpu.VMEM((CH,), jnp.int32),
        pltpu.VMEM((CH,), jnp.float32),
        pltpu.VMEM((RPT,), jnp.float32),
        pltpu.VMEM_SHARED((NPAD,), jnp.float32),
        pltpu.VMEM_SHARED((NPAD,), jnp.float32),
    ),
)
def _degrees(src_hbm, dst_hbm, dsrc_out, ddst_out,
             sbuf, dbuf, ones, zbuf, dsrc_sp, ddst_sp):
    c = lax.axis_index("c")
    s = lax.axis_index("s")
    wid = s * NC + c
    base = wid * EPW

    def _zero(i, _):
        zbuf[pl.ds(i * 16, 16)] = jnp.zeros((16,), jnp.float32)
        return _
    lax.fori_loop(0, RPT // 16, _zero, None)

    def _ones(i, _):
        ones[pl.ds(i * 16, 16)] = jnp.ones((16,), jnp.float32)
        return _
    lax.fori_loop(0, CH // 16, _ones, None)

    pltpu.sync_copy(zbuf, dsrc_sp.at[pl.ds(s * RPT, RPT)])
    pltpu.sync_copy(zbuf, ddst_sp.at[pl.ds(s * RPT, RPT)])
    plsc.subcore_barrier()

    def _body(i, _):
        off = base + i * CH
        pltpu.sync_copy(src_hbm.at[pl.ds(off, CH)], sbuf)
        pltpu.sync_copy(dst_hbm.at[pl.ds(off, CH)], dbuf)
        pltpu.sync_copy(ones, dsrc_sp.at[sbuf], add=True)
        pltpu.sync_copy(ones, ddst_sp.at[dbuf], add=True)
        return _
    lax.fori_loop(0, NCHUNK, _body, None)
    plsc.subcore_barrier()

    pltpu.sync_copy(dsrc_sp.at[pl.ds(s * RPT, RPT)],
                    dsrc_out.at[c].at[pl.ds(s * RPT, RPT)])
    pltpu.sync_copy(ddst_sp.at[pl.ds(s * RPT, RPT)],
                    ddst_out.at[c].at[pl.ds(s * RPT, RPT)])


def _mm_body(x_ref, w_ref, dso_ref, dsi_ref, h_ref, rout_ref, rin_ref):
    do = dso_ref[0] + dso_ref[1]
    di = dsi_ref[0] + dsi_ref[1]
    ro = lax.rsqrt(jnp.maximum(do, 1.0))
    ri = lax.rsqrt(jnp.maximum(di, 1.0))
    rout_ref[...] = ro
    rin_ref[...] = ri
    h_ref[...] = jnp.dot(x_ref[...] * ro, w_ref[...],
                         preferred_element_type=jnp.float32)


def _matmul(x_pad, W1, dsrc_p, ddst_p):
    return pl.pallas_call(
        _mm_body,
        grid=(GR,),
        in_specs=[
            pl.BlockSpec((BR, F), lambda i: (i, 0)),
            pl.BlockSpec((F, F), lambda i: (0, 0)),
            pl.BlockSpec((NC, BR, 1), lambda i: (0, i, 0)),
            pl.BlockSpec((NC, BR, 1), lambda i: (0, i, 0)),
        ],
        out_specs=[
            pl.BlockSpec((BR, F), lambda i: (i, 0)),
            pl.BlockSpec((BR, 1), lambda i: (i, 0)),
            pl.BlockSpec((BR, 1), lambda i: (i, 0)),
        ],
        out_shape=[
            jax.ShapeDtypeStruct((NPAD, F), jnp.float32),
            jax.ShapeDtypeStruct((NPAD, 1), jnp.float32),
            jax.ShapeDtypeStruct((NPAD, 1), jnp.float32),
        ],
    )(x_pad, W1, dsrc_p, ddst_p)


@functools.partial(
    pl.kernel,
    out_type=(jax.ShapeDtypeStruct((NC, NPAD, F), jnp.float32),
              jax.ShapeDtypeStruct((NC, NPAD), jnp.float32)),
    mesh=_MESH,
    scratch_types=(
        pltpu.VMEM((CH,), jnp.int32),
        pltpu.VMEM((CH,), jnp.int32),
        pltpu.VMEM((CH, F), jnp.float32),
        pltpu.VMEM((CH,), jnp.float32),
        pltpu.VMEM((CH, F), jnp.float32),
        pltpu.VMEM_SHARED((NPAD, F), jnp.float32),
        pltpu.VMEM_SHARED((NPAD,), jnp.float32),
    ),
)
def _propagate(h_hbm, src_hbm, dst_hbm, rin_hbm, agg_out, c_out,
               sbuf, dbuf, rows, rvals, zrows, agg_sp, c_sp):
    c = lax.axis_index("c")
    s = lax.axis_index("s")
    wid = s * NC + c
    base = wid * EPW

    def _zero(i, _):
        r = i // (F // 16)
        k = i % (F // 16)
        zrows[r, pl.ds(k * 16, 16)] = jnp.zeros((16,), jnp.float32)
        return _
    lax.fori_loop(0, CH * (F // 16), _zero, None)

    for k in range(RPT // CH):
        pltpu.sync_copy(zrows, agg_sp.at[pl.ds(s * RPT + k * CH, CH)])
    for k in range(RPT // F):
        pltpu.sync_copy(zrows.at[0], c_sp.at[pl.ds(s * RPT + k * F, F)])
    plsc.subcore_barrier()

    def _body(i, _):
        off = base + i * CH
        pltpu.sync_copy(src_hbm.at[pl.ds(off, CH)], sbuf)
        pltpu.sync_copy(dst_hbm.at[pl.ds(off, CH)], dbuf)
        pltpu.sync_copy(h_hbm.at[sbuf], rows)
        pltpu.sync_copy(rows, agg_sp.at[dbuf], add=True)
        pltpu.sync_copy(rin_hbm.at[dbuf], rvals)
        pltpu.sync_copy(rvals, c_sp.at[sbuf], add=True)
        return _
    lax.fori_loop(0, NCHUNK, _body, None)
    plsc.subcore_barrier()

    pltpu.sync_copy(agg_sp.at[pl.ds(s * RPT, RPT)],
                    agg_out.at[c].at[pl.ds(s * RPT, RPT)])
    pltpu.sync_copy(c_sp.at[pl.ds(s * RPT, RPT)],
                    c_out.at[c].at[pl.ds(s * RPT, RPT)])


def _fin_body(agg_ref, c_ref, rin_ref, rout_ref, b1_ref, w2_ref, b2_ref,
              out_ref, acc_ref):
    i = pl.program_id(0)
    a = agg_ref[0] + agg_ref[1]
    g = a * rin_ref[...] + b1_ref[...]
    h1 = jnp.maximum(g, 0.0)
    w = (c_ref[0] + c_ref[1]) * rout_ref[...]
    part = jnp.sum((h1 * w).reshape(BR // 8, 8, F), axis=0)

    @pl.when(i == 0)
    def _():
        acc_ref[...] = part

    @pl.when(i > 0)
    def _():
        acc_ref[...] = acc_ref[...] + part

    @pl.when(i == GR - 1)
    def _():
        sv = jnp.sum(acc_ref[...], axis=0, keepdims=True)
        out_ref[...] = (jnp.dot(sv, w2_ref[...],
                                preferred_element_type=jnp.float32)
                        * (1.0 / N) + b2_ref[...])


def _finalize(agg_p, c_p, r_in, r_out, b1, W2, b2):
    return pl.pallas_call(
        _fin_body,
        grid=(GR,),
        in_specs=[
            pl.BlockSpec((NC, BR, F), lambda i: (0, i, 0)),
            pl.BlockSpec((NC, BR, 1), lambda i: (0, i, 0)),
            pl.BlockSpec((BR, 1), lambda i: (i, 0)),
            pl.BlockSpec((BR, 1), lambda i: (i, 0)),
            pl.BlockSpec((1, F), lambda i: (0, 0)),
            pl.BlockSpec((F, C), lambda i: (0, 0)),
            pl.BlockSpec((1, C), lambda i: (0, 0)),
        ],
        out_specs=pl.BlockSpec((1, C), lambda i: (0, 0)),
        out_shape=jax.ShapeDtypeStruct((1, C), jnp.float32),
        scratch_shapes=[pltpu.VMEM((8, F), jnp.float32)],
    )(agg_p, c_p, r_in, r_out, b1, W2, b2)


def kernel(x, edge_index, W1, b1, W2, b2):
    src = edge_index[0].astype(jnp.int32)
    dst = edge_index[1].astype(jnp.int32)
    x_pad = jnp.zeros((NPAD, F), jnp.float32).at[:N].set(x)
    dsrc_p, ddst_p = _degrees(src, dst)
    h, r_out, r_in = _matmul(x_pad, W1,
                             dsrc_p.reshape(NC, NPAD, 1),
                             ddst_p.reshape(NC, NPAD, 1))
    agg_p, c_p = _propagate(h, src, dst, r_in.reshape(NPAD))
    out = _finalize(agg_p, c_p.reshape(NC, NPAD, 1), r_in, r_out,
                    b1.reshape(1, F), W2, b2.reshape(1, C))
    return out


# speedup vs baseline: 11.0493x; 1.9840x over previous
import functools

import jax
import jax.numpy as jnp
from jax import lax
from jax.experimental import pallas as pl
from jax.experimental.pallas import tpu as pltpu
from jax.experimental.pallas import tpu_sc as plsc

N = 10000
E = 320000
NPAD = 10240
F = 128
C = 40
NC = 2
NS = 16
NW = NC * NS
EPW = E // NW
CH = 80
NCHUNK = EPW // CH
RPT = NPAD // NS
BR = 1024
GR = NPAD // BR
DEPTH = 4

_MESH = plsc.VectorSubcoreMesh(
    core_axis_name="c", subcore_axis_name="s", num_cores=NC, num_subcores=NS)


@functools.partial(
    pl.kernel,
    out_type=(jax.ShapeDtypeStruct((NC, NPAD), jnp.float32),
              jax.ShapeDtypeStruct((NC, NPAD), jnp.float32)),
    mesh=_MESH,
    scratch_types=(
        pltpu.VMEM((NCHUNK, CH), jnp.int32),
        pltpu.VMEM((NCHUNK, CH), jnp.int32),
        pltpu.VMEM((CH,), jnp.float32),
        pltpu.VMEM((RPT,), jnp.float32),
        pltpu.VMEM_SHARED((NPAD,), jnp.float32),
        pltpu.VMEM_SHARED((NPAD,), jnp.float32),
        pltpu.SemaphoreType.DMA,
        pltpu.SemaphoreType.DMA,
    ),
)
def _degrees(src_hbm, dst_hbm, dsrc_out, ddst_out,
             sidx, didx, ones, zbuf, dsrc_sp, ddst_sp, sem_s, sem_d):
    c = lax.axis_index("c")
    s = lax.axis_index("s")
    wid = s * NC + c

    def _zero(i, _):
        zbuf[pl.ds(i * 16, 16)] = jnp.zeros((16,), jnp.float32)
        return _
    lax.fori_loop(0, RPT // 16, _zero, None)

    def _ones(i, _):
        ones[pl.ds(i * 16, 16)] = jnp.ones((16,), jnp.float32)
        return _
    lax.fori_loop(0, CH // 16, _ones, None)

    pltpu.sync_copy(src_hbm.at[wid], sidx)
    pltpu.sync_copy(dst_hbm.at[wid], didx)
    pltpu.sync_copy(zbuf, dsrc_sp.at[pl.ds(s * RPT, RPT)])
    pltpu.sync_copy(zbuf, ddst_sp.at[pl.ds(s * RPT, RPT)])
    plsc.subcore_barrier()

    def _body(i, _):
        pltpu.async_copy(ones, dsrc_sp.at[sidx.at[i]], sem_s, add=True)
        pltpu.async_copy(ones, ddst_sp.at[didx.at[i]], sem_d, add=True)

        @pl.when(i >= DEPTH)
        def _w():
            j = i - DEPTH
            pltpu.make_async_copy(ones, dsrc_sp.at[sidx.at[j]], sem_s).wait()
            pltpu.make_async_copy(ones, ddst_sp.at[didx.at[j]], sem_d).wait()
        return _
    lax.fori_loop(0, NCHUNK, _body, None)

    def _drain(i, _):
        j = NCHUNK - DEPTH + i
        pltpu.make_async_copy(ones, dsrc_sp.at[sidx.at[j]], sem_s).wait()
        pltpu.make_async_copy(ones, ddst_sp.at[didx.at[j]], sem_d).wait()
        return _
    lax.fori_loop(0, DEPTH, _drain, None)
    plsc.subcore_barrier()

    pltpu.sync_copy(dsrc_sp.at[pl.ds(s * RPT, RPT)],
                    dsrc_out.at[c].at[pl.ds(s * RPT, RPT)])
    pltpu.sync_copy(ddst_sp.at[pl.ds(s * RPT, RPT)],
                    ddst_out.at[c].at[pl.ds(s * RPT, RPT)])


def _mm_body(x_ref, w_ref, dso_ref, dsi_ref, h_ref, rout_ref, rin_ref):
    do = dso_ref[0] + dso_ref[1]
    di = dsi_ref[0] + dsi_ref[1]
    ro = lax.rsqrt(jnp.maximum(do, 1.0))
    ri = lax.rsqrt(jnp.maximum(di, 1.0))
    rout_ref[...] = ro
    rin_ref[...] = ri
    h_ref[...] = jnp.dot(x_ref[...] * ro, w_ref[...],
                         preferred_element_type=jnp.float32)


def _matmul(x_pad, W1, dsrc_p, ddst_p):
    return pl.pallas_call(
        _mm_body,
        grid=(GR,),
        in_specs=[
            pl.BlockSpec((BR, F), lambda i: (i, 0)),
            pl.BlockSpec((F, F), lambda i: (0, 0)),
            pl.BlockSpec((NC, BR, 1), lambda i: (0, i, 0)),
            pl.BlockSpec((NC, BR, 1), lambda i: (0, i, 0)),
        ],
        out_specs=[
            pl.BlockSpec((BR, F), lambda i: (i, 0)),
            pl.BlockSpec((BR, 1), lambda i: (i, 0)),
            pl.BlockSpec((BR, 1), lambda i: (i, 0)),
        ],
        out_shape=[
            jax.ShapeDtypeStruct((NPAD, F), jnp.float32),
            jax.ShapeDtypeStruct((NPAD, 1), jnp.float32),
            jax.ShapeDtypeStruct((NPAD, 1), jnp.float32),
        ],
    )(x_pad, W1, dsrc_p, ddst_p)


@functools.partial(
    pl.kernel,
    out_type=(jax.ShapeDtypeStruct((NC, NPAD, F), jnp.float32),
              jax.ShapeDtypeStruct((NC, NPAD), jnp.float32)),
    mesh=_MESH,
    scratch_types=(
        pltpu.VMEM((NCHUNK, CH), jnp.int32),
        pltpu.VMEM((NCHUNK, CH), jnp.int32),
        pltpu.VMEM((CH, F), jnp.float32),
        pltpu.VMEM((CH,), jnp.float32),
        pltpu.VMEM((CH,), jnp.float32),
        pltpu.VMEM_SHARED((NPAD, F), jnp.float32),
        pltpu.VMEM_SHARED((NPAD,), jnp.float32),
        pltpu.SemaphoreType.DMA,
        pltpu.SemaphoreType.DMA,
    ),
)
def _propagate(h_hbm, src_hbm, dst_hbm, rin_hbm, agg_out, c_out,
               sidx, didx, rows, rvals0, rvals1,
               agg_sp, c_sp, rsem0, rsem1):
    c = lax.axis_index("c")
    s = lax.axis_index("s")
    wid = s * NC + c

    def _zero(i, _):
        r = i // (F // 16)
        k = i % (F // 16)
        rows[r, pl.ds(k * 16, 16)] = jnp.zeros((16,), jnp.float32)
        return _
    lax.fori_loop(0, CH * (F // 16), _zero, None)

    pltpu.sync_copy(src_hbm.at[wid], sidx)
    pltpu.sync_copy(dst_hbm.at[wid], didx)
    for k in range(RPT // CH):
        pltpu.sync_copy(rows, agg_sp.at[pl.ds(s * RPT + k * CH, CH)])
    for k in range(RPT // F):
        pltpu.sync_copy(rows.at[0], c_sp.at[pl.ds(s * RPT + k * F, F)])

    plsc.subcore_barrier()

    rbufs = ((rvals0, rsem0), (rvals1, rsem1))

    def _start_rgather(j, b):
        rvals, rsem = rbufs[b]
        pltpu.async_copy(rin_hbm.at[didx.at[j]], rvals, rsem)

    def _wait_rgather(j, b):
        rvals, rsem = rbufs[b]
        pltpu.make_async_copy(rin_hbm.at[didx.at[j]], rvals, rsem).wait()

    def _start_rscatter(j, b):
        rvals, rsem = rbufs[b]
        pltpu.async_copy(rvals, c_sp.at[sidx.at[j]], rsem, add=True)

    def _wait_rscatter(j, b):
        rvals, rsem = rbufs[b]
        pltpu.make_async_copy(rvals, c_sp.at[sidx.at[j]], rsem).wait()

    def _rows_chunk(j):
        pltpu.sync_copy(h_hbm.at[sidx.at[j]], rows)
        pltpu.sync_copy(rows, agg_sp.at[didx.at[j]], add=True)

    _start_rgather(0, 0)

    def _body(i, _):
        j = i * 2
        _rows_chunk(j)
        _wait_rgather(j, 0)

        @pl.when(i > 0)
        def _():
            _wait_rscatter(j - 1, 1)
        _start_rgather(j + 1, 1)
        _start_rscatter(j, 0)
        _rows_chunk(j + 1)
        _wait_rgather(j + 1, 1)
        _wait_rscatter(j, 0)
        _start_rgather(j + 2, 0)
        _start_rscatter(j + 1, 1)
        return _
    lax.fori_loop(0, (NCHUNK - 1) // 2, _body, None)

    _rows_chunk(NCHUNK - 1)
    _wait_rgather(NCHUNK - 1, 0)
    _wait_rscatter(NCHUNK - 2, 1)
    _start_rscatter(NCHUNK - 1, 0)
    _wait_rscatter(NCHUNK - 1, 0)
    plsc.subcore_barrier()

    pltpu.sync_copy(agg_sp.at[pl.ds(s * RPT, RPT)],
                    agg_out.at[c].at[pl.ds(s * RPT, RPT)])
    pltpu.sync_copy(c_sp.at[pl.ds(s * RPT, RPT)],
                    c_out.at[c].at[pl.ds(s * RPT, RPT)])


def _fin_body(agg_ref, c_ref, rin_ref, rout_ref, b1_ref, w2_ref, b2_ref,
              out_ref, acc_ref):
    i = pl.program_id(0)
    a = agg_ref[0] + agg_ref[1]
    g = a * rin_ref[...] + b1_ref[...]
    h1 = jnp.maximum(g, 0.0)
    row = jax.lax.broadcasted_iota(jnp.int32, (BR, 1), 0) + i * BR
    w = jnp.where(row < N, (c_ref[0] + c_ref[1]) * rout_ref[...], 0.0)
    part = jnp.sum((h1 * w).reshape(BR // 8, 8, F), axis=0)

    @pl.when(i == 0)
    def _():
        acc_ref[...] = part

    @pl.when(i > 0)
    def _():
        acc_ref[...] = acc_ref[...] + part

    @pl.when(i == GR - 1)
    def _():
        sv = jnp.sum(acc_ref[...], axis=0, keepdims=True)
        out_ref[...] = (jnp.dot(sv, w2_ref[...],
                                preferred_element_type=jnp.float32)
                        * (1.0 / N) + b2_ref[...])


def _finalize(agg_p, c_p, r_in, r_out, b1, W2, b2):
    return pl.pallas_call(
        _fin_body,
        grid=(GR,),
        in_specs=[
            pl.BlockSpec((NC, BR, F), lambda i: (0, i, 0)),
            pl.BlockSpec((NC, BR, 1), lambda i: (0, i, 0)),
            pl.BlockSpec((BR, 1), lambda i: (i, 0)),
            pl.BlockSpec((BR, 1), lambda i: (i, 0)),
            pl.BlockSpec((1, F), lambda i: (0, 0)),
            pl.BlockSpec((F, C), lambda i: (0, 0)),
            pl.BlockSpec((1, C), lambda i: (0, 0)),
        ],
        out_specs=pl.BlockSpec((1, C), lambda i: (0, 0)),
        out_shape=jax.ShapeDtypeStruct((1, C), jnp.float32),
        scratch_shapes=[pltpu.VMEM((8, F), jnp.float32)],
    )(agg_p, c_p, r_in, r_out, b1, W2, b2)


def kernel(x, edge_index, W1, b1, W2, b2):
    src = edge_index[0].astype(jnp.int32).reshape(NW, NCHUNK, CH)
    dst = edge_index[1].astype(jnp.int32).reshape(NW, NCHUNK, CH)
    x_pad = jnp.zeros((NPAD, F), jnp.float32).at[:N].set(x)
    dsrc_p, ddst_p = _degrees(src, dst)
    h, r_out, r_in = _matmul(x_pad, W1,
                             dsrc_p.reshape(NC, NPAD, 1),
                             ddst_p.reshape(NC, NPAD, 1))
    agg_p, c_p = _propagate(h, src, dst, r_in.reshape(NPAD))
    out = _finalize(agg_p, c_p.reshape(NC, NPAD, 1), r_in, r_out,
                    b1.reshape(1, F), W2, b2.reshape(1, C))
    return out


# speedup vs baseline: 14.2512x; 1.2898x over previous
import functools

import jax
import jax.numpy as jnp
from jax import lax
from jax.experimental import pallas as pl
from jax.experimental.pallas import tpu as pltpu
from jax.experimental.pallas import tpu_sc as plsc

N = 10000
E = 320000
NPAD = 10240
F = 128
C = 40
NC = 2
NS = 16
NW = NC * NS
EPW = E // NW
CH1 = 80
NCH1 = EPW // CH1
DEPTH = 4
CH = 128
NCHUNK = 79
EPW_PAD = NCHUNK * CH
PH0 = 40
RPT = NPAD // NS
BR = 1024
GR = NPAD // BR

_MESH = plsc.VectorSubcoreMesh(
    core_axis_name="c", subcore_axis_name="s", num_cores=NC, num_subcores=NS)


@functools.partial(
    pl.kernel,
    out_type=(jax.ShapeDtypeStruct((NC, NPAD), jnp.float32),
              jax.ShapeDtypeStruct((NC, NPAD), jnp.float32)),
    mesh=_MESH,
    scratch_types=(
        pltpu.VMEM((NCH1, CH1), jnp.int32),
        pltpu.VMEM((NCH1, CH1), jnp.int32),
        pltpu.VMEM((CH1,), jnp.float32),
        pltpu.VMEM((RPT,), jnp.float32),
        pltpu.VMEM_SHARED((NPAD,), jnp.float32),
        pltpu.VMEM_SHARED((NPAD,), jnp.float32),
        pltpu.SemaphoreType.DMA,
        pltpu.SemaphoreType.DMA,
    ),
)
def _degrees(src_hbm, dst_hbm, dsrc_out, ddst_out,
             sidx, didx, ones, zbuf, dsrc_sp, ddst_sp, sem_s, sem_d):
    c = lax.axis_index("c")
    s = lax.axis_index("s")
    wid = s * NC + c

    def _zero(i, _):
        zbuf[pl.ds(i * 16, 16)] = jnp.zeros((16,), jnp.float32)
        return _
    lax.fori_loop(0, RPT // 16, _zero, None)

    def _ones(i, _):
        ones[pl.ds(i * 16, 16)] = jnp.ones((16,), jnp.float32)
        return _
    lax.fori_loop(0, CH1 // 16, _ones, None)

    pltpu.sync_copy(src_hbm.at[wid], sidx)
    pltpu.sync_copy(dst_hbm.at[wid], didx)
    pltpu.sync_copy(zbuf, dsrc_sp.at[pl.ds(s * RPT, RPT)])
    pltpu.sync_copy(zbuf, ddst_sp.at[pl.ds(s * RPT, RPT)])
    plsc.subcore_barrier()

    def _body(i, _):
        pltpu.async_copy(ones, dsrc_sp.at[sidx.at[i]], sem_s, add=True)
        pltpu.async_copy(ones, ddst_sp.at[didx.at[i]], sem_d, add=True)

        @pl.when(i >= DEPTH)
        def _w():
            j = i - DEPTH
            pltpu.make_async_copy(ones, dsrc_sp.at[sidx.at[j]], sem_s).wait()
            pltpu.make_async_copy(ones, ddst_sp.at[didx.at[j]], sem_d).wait()
        return _
    lax.fori_loop(0, NCH1, _body, None)

    def _drain(i, _):
        j = NCH1 - DEPTH + i
        pltpu.make_async_copy(ones, dsrc_sp.at[sidx.at[j]], sem_s).wait()
        pltpu.make_async_copy(ones, ddst_sp.at[didx.at[j]], sem_d).wait()
        return _
    lax.fori_loop(0, DEPTH, _drain, None)
    plsc.subcore_barrier()

    pltpu.sync_copy(dsrc_sp.at[pl.ds(s * RPT, RPT)],
                    dsrc_out.at[c].at[pl.ds(s * RPT, RPT)])
    pltpu.sync_copy(ddst_sp.at[pl.ds(s * RPT, RPT)],
                    ddst_out.at[c].at[pl.ds(s * RPT, RPT)])


def _mm_body(x_ref, w_ref, dso_ref, dsi_ref, h_ref, rout_ref, rin_ref):
    do = dso_ref[0] + dso_ref[1]
    di = dsi_ref[0] + dsi_ref[1]
    ro = lax.rsqrt(jnp.maximum(do, 1.0))
    ri = lax.rsqrt(jnp.maximum(di, 1.0))
    rout_ref[...] = ro
    rin_ref[...] = ri
    h_ref[...] = jnp.dot(x_ref[...] * ro, w_ref[...],
                         preferred_element_type=jnp.float32)


def _matmul(x_pad, W1, dsrc_p, ddst_p):
    return pl.pallas_call(
        _mm_body,
        grid=(GR,),
        in_specs=[
            pl.BlockSpec((BR, F), lambda i: (i, 0)),
            pl.BlockSpec((F, F), lambda i: (0, 0)),
            pl.BlockSpec((NC, BR, 1), lambda i: (0, i, 0)),
            pl.BlockSpec((NC, BR, 1), lambda i: (0, i, 0)),
        ],
        out_specs=[
            pl.BlockSpec((BR, F), lambda i: (i, 0)),
            pl.BlockSpec((BR, 1), lambda i: (i, 0)),
            pl.BlockSpec((BR, 1), lambda i: (i, 0)),
        ],
        out_shape=[
            jax.ShapeDtypeStruct((NPAD, F), jnp.float32),
            jax.ShapeDtypeStruct((NPAD, 1), jnp.float32),
            jax.ShapeDtypeStruct((NPAD, 1), jnp.float32),
        ],
    )(x_pad, W1, dsrc_p, ddst_p)


@functools.partial(
    pl.kernel,
    out_type=(jax.ShapeDtypeStruct((NC, NPAD, F), jnp.float32),
              jax.ShapeDtypeStruct((NC, NPAD), jnp.float32)),
    mesh=_MESH,
    scratch_types=(
        pltpu.VMEM((PH0, CH), jnp.int32),
        pltpu.VMEM((PH0, CH), jnp.int32),
        pltpu.VMEM((CH, F), jnp.float32),
        pltpu.VMEM((CH, F), jnp.float32),
        pltpu.VMEM((CH,), jnp.float32),
        pltpu.VMEM((CH,), jnp.float32),
        pltpu.VMEM_SHARED((NPAD, F), jnp.float32),
        pltpu.VMEM_SHARED((NPAD,), jnp.float32),
        pltpu.SemaphoreType.DMA,
        pltpu.SemaphoreType.DMA,
        pltpu.SemaphoreType.DMA,
        pltpu.SemaphoreType.DMA,
        pltpu.SemaphoreType.DMA,
        pltpu.SemaphoreType.DMA,
        pltpu.SemaphoreType.DMA,
        pltpu.SemaphoreType.DMA,
    ),
)
def _propagate(h_hbm, src_hbm, dst_hbm, rin_hbm, agg_out, c_out,
               sidx, didx, rows0, rows1, rvals0, rvals1,
               agg_sp, c_sp, gsem0, gsem1, ssem0, ssem1,
               rgsem0, rgsem1, rssem0, rssem1):
    c = lax.axis_index("c")
    s = lax.axis_index("s")
    wid = s * NC + c

    def _zero(i, _):
        r = i // (F // 16)
        k = i % (F // 16)
        rows0[r, pl.ds(k * 16, 16)] = jnp.zeros((16,), jnp.float32)
        return _
    lax.fori_loop(0, CH * (F // 16), _zero, None)

    for k in range(RPT // CH):
        pltpu.sync_copy(rows0, agg_sp.at[pl.ds(s * RPT + k * CH, CH)])
    for k in range(RPT // F):
        pltpu.sync_copy(rows0.at[0], c_sp.at[pl.ds(s * RPT + k * F, F)])

    plsc.subcore_barrier()

    rbufs = ((rows0, gsem0, ssem0), (rows1, gsem1, ssem1))
    cbufs = ((rvals0, rgsem0, rssem0), (rvals1, rgsem1, rssem1))

    def _start_gather(j, b):
        rows, gsem, _ = rbufs[b]
        pltpu.async_copy(h_hbm.at[sidx.at[j]], rows, gsem)

    def _wait_gather(j, b):
        rows, gsem, _ = rbufs[b]
        pltpu.make_async_copy(h_hbm.at[sidx.at[j]], rows, gsem).wait()

    def _start_scatter(j, b):
        rows, _, ssem = rbufs[b]
        pltpu.async_copy(rows, agg_sp.at[didx.at[j]], ssem, add=True)

    def _wait_scatter(j, b):
        rows, _, ssem = rbufs[b]
        pltpu.make_async_copy(rows, agg_sp.at[didx.at[j]], ssem).wait()

    def _start_rgather(j, b):
        rvals, rgsem, _ = cbufs[b]
        pltpu.async_copy(rin_hbm.at[didx.at[j]], rvals, rgsem)

    def _wait_rgather(j, b):
        rvals, rgsem, _ = cbufs[b]
        pltpu.make_async_copy(rin_hbm.at[didx.at[j]], rvals, rgsem).wait()

    def _start_cscatter(j, b):
        rvals, _, rssem = cbufs[b]
        pltpu.async_copy(rvals, c_sp.at[sidx.at[j]], rssem, add=True)

    def _wait_cscatter(j, b):
        rvals, _, rssem = cbufs[b]
        pltpu.make_async_copy(rvals, c_sp.at[sidx.at[j]], rssem).wait()

    def _phase(glo, cnt):
        pltpu.sync_copy(src_hbm.at[wid].at[pl.ds(glo, cnt)],
                        sidx.at[pl.ds(0, cnt)])
        pltpu.sync_copy(dst_hbm.at[wid].at[pl.ds(glo, cnt)],
                        didx.at[pl.ds(0, cnt)])

        _start_gather(0, 0)

        def _body(i, _):
            j = i * 2
            _wait_gather(j, 0)

            @pl.when(i > 0)
            def _():
                _wait_scatter(j - 1, 1)
            _start_gather(j + 1, 1)
            _start_scatter(j, 0)

            @pl.when(i > 0)
            def _():
                _wait_cscatter(j - 2, 0)
                _wait_cscatter(j - 1, 1)
            _start_rgather(j, 0)
            _start_rgather(j + 1, 1)

            _wait_gather(j + 1, 1)
            _wait_scatter(j, 0)
            _start_gather(j + 2, 0)
            _start_scatter(j + 1, 1)

            _wait_rgather(j, 0)
            _start_cscatter(j, 0)
            _wait_rgather(j + 1, 1)
            _start_cscatter(j + 1, 1)
            return _
        nb = (cnt - 1) // 2
        lax.fori_loop(0, nb, _body, None)

        if cnt % 2 == 0:
            _wait_gather(cnt - 2, 0)
            _wait_scatter(cnt - 3, 1)
            _start_gather(cnt - 1, 1)
            _start_scatter(cnt - 2, 0)
            _wait_gather(cnt - 1, 1)
            _wait_scatter(cnt - 2, 0)
            _start_scatter(cnt - 1, 1)
            _wait_scatter(cnt - 1, 1)
            _wait_cscatter(cnt - 4, 0)
            _wait_cscatter(cnt - 3, 1)
            _start_rgather(cnt - 2, 0)
            _start_rgather(cnt - 1, 1)
            _wait_rgather(cnt - 2, 0)
            _start_cscatter(cnt - 2, 0)
            _wait_rgather(cnt - 1, 1)
            _start_cscatter(cnt - 1, 1)
            _wait_cscatter(cnt - 2, 0)
            _wait_cscatter(cnt - 1, 1)
        else:
            _wait_gather(cnt - 1, 0)
            _wait_scatter(cnt - 2, 1)
            _start_scatter(cnt - 1, 0)
            _wait_scatter(cnt - 1, 0)
            _wait_cscatter(cnt - 3, 0)
            _wait_cscatter(cnt - 2, 1)
            _start_rgather(cnt - 1, 0)
            _wait_rgather(cnt - 1, 0)
            _start_cscatter(cnt - 1, 0)
            _wait_cscatter(cnt - 1, 0)

    _phase(0, PH0)
    _phase(PH0, NCHUNK - PH0)
    plsc.subcore_barrier()

    pltpu.sync_copy(agg_sp.at[pl.ds(s * RPT, RPT)],
                    agg_out.at[c].at[pl.ds(s * RPT, RPT)])
    pltpu.sync_copy(c_sp.at[pl.ds(s * RPT, RPT)],
                    c_out.at[c].at[pl.ds(s * RPT, RPT)])


def _fin_body(agg_ref, c_ref, rin_ref, rout_ref, b1_ref, w2_ref, b2_ref,
              out_ref, acc_ref):
    i = pl.program_id(0)
    a = agg_ref[0] + agg_ref[1]
    g = a * rin_ref[...] + b1_ref[...]
    h1 = jnp.maximum(g, 0.0)
    row = jax.lax.broadcasted_iota(jnp.int32, (BR, 1), 0) + i * BR
    w = jnp.where(row < N, (c_ref[0] + c_ref[1]) * rout_ref[...], 0.0)
    part = jnp.sum((h1 * w).reshape(BR // 8, 8, F), axis=0)

    @pl.when(i == 0)
    def _():
        acc_ref[...] = part

    @pl.when(i > 0)
    def _():
        acc_ref[...] = acc_ref[...] + part

    @pl.when(i == GR - 1)
    def _():
        sv = jnp.sum(acc_ref[...], axis=0, keepdims=True)
        out_ref[...] = (jnp.dot(sv, w2_ref[...],
                                preferred_element_type=jnp.float32)
                        * (1.0 / N) + b2_ref[...])


def _finalize(agg_p, c_p, r_in, r_out, b1, W2, b2):
    return pl.pallas_call(
        _fin_body,
        grid=(GR,),
        in_specs=[
            pl.BlockSpec((NC, BR, F), lambda i: (0, i, 0)),
            pl.BlockSpec((NC, BR, 1), lambda i: (0, i, 0)),
            pl.BlockSpec((BR, 1), lambda i: (i, 0)),
            pl.BlockSpec((BR, 1), lambda i: (i, 0)),
            pl.BlockSpec((1, F), lambda i: (0, 0)),
            pl.BlockSpec((F, C), lambda i: (0, 0)),
            pl.BlockSpec((1, C), lambda i: (0, 0)),
        ],
        out_specs=pl.BlockSpec((1, C), lambda i: (0, 0)),
        out_shape=jax.ShapeDtypeStruct((1, C), jnp.float32),
        scratch_shapes=[pltpu.VMEM((8, F), jnp.float32)],
    )(agg_p, c_p, r_in, r_out, b1, W2, b2)


def kernel(x, edge_index, W1, b1, W2, b2):
    src = edge_index[0].astype(jnp.int32)
    dst = edge_index[1].astype(jnp.int32)
    src32 = src.reshape(NW, NCH1, CH1)
    dst32 = dst.reshape(NW, NCH1, CH1)
    npad_e = EPW_PAD - EPW
    pad_rows = (jnp.arange(NW * npad_e, dtype=jnp.int32).reshape(NW, npad_e)
                % (NPAD - N)) + N
    srcp = jnp.concatenate(
        [src.reshape(NW, EPW), pad_rows], axis=1).reshape(NW, NCHUNK, CH)
    dstp = jnp.concatenate(
        [dst.reshape(NW, EPW), pad_rows], axis=1).reshape(NW, NCHUNK, CH)
    x_pad = jnp.zeros((NPAD, F), jnp.float32).at[:N].set(x)
    dsrc_p, ddst_p = _degrees(src32, dst32)
    h, r_out, r_in = _matmul(x_pad, W1,
                             dsrc_p.reshape(NC, NPAD, 1),
                             ddst_p.reshape(NC, NPAD, 1))
    agg_p, c_p = _propagate(h, srcp, dstp, r_in.reshape(NPAD))
    out = _finalize(agg_p, c_p.reshape(NC, NPAD, 1), r_in, r_out,
                    b1.reshape(1, F), W2, b2.reshape(1, C))
    return out


# speedup vs baseline: 14.7851x; 1.0375x over previous
import functools

import jax
import jax.numpy as jnp
from jax import lax
from jax.experimental import pallas as pl
from jax.experimental.pallas import tpu as pltpu
from jax.experimental.pallas import tpu_sc as plsc

N = 10000
E = 320000
NPAD = 10240
F = 128
C = 40
NC = 2
NS = 16
NW = NC * NS
EPW = E // NW
CH1 = 80
NCH1 = EPW // CH1
DEPTH = 4
CH = 128
NCHUNK = 79
EPW_PAD = NCHUNK * CH
PH0 = 40
RPT = NPAD // NS
BR = 1024
GR = NPAD // BR

_MESH = plsc.VectorSubcoreMesh(
    core_axis_name="c", subcore_axis_name="s", num_cores=NC, num_subcores=NS)


@functools.partial(
    pl.kernel,
    out_type=(jax.ShapeDtypeStruct((NC, NPAD), jnp.float32),
              jax.ShapeDtypeStruct((NC, NPAD), jnp.float32)),
    mesh=_MESH,
    scratch_types=(
        pltpu.VMEM((NCH1, CH1), jnp.int32),
        pltpu.VMEM((NCH1, CH1), jnp.int32),
        pltpu.VMEM((CH1,), jnp.float32),
        pltpu.VMEM((RPT,), jnp.float32),
        pltpu.VMEM_SHARED((NPAD,), jnp.float32),
        pltpu.VMEM_SHARED((NPAD,), jnp.float32),
        pltpu.SemaphoreType.DMA,
        pltpu.SemaphoreType.DMA,
    ),
)
def _degrees(src_hbm, dst_hbm, dsrc_out, ddst_out,
             sidx, didx, ones, zbuf, dsrc_sp, ddst_sp, sem_s, sem_d):
    c = lax.axis_index("c")
    s = lax.axis_index("s")
    wid = s * NC + c

    def _zero(i, _):
        zbuf[pl.ds(i * 16, 16)] = jnp.zeros((16,), jnp.float32)
        return _
    lax.fori_loop(0, RPT // 16, _zero, None)

    def _ones(i, _):
        ones[pl.ds(i * 16, 16)] = jnp.ones((16,), jnp.float32)
        return _
    lax.fori_loop(0, CH1 // 16, _ones, None)

    pltpu.sync_copy(src_hbm.at[wid], sidx)
    pltpu.sync_copy(dst_hbm.at[wid], didx)
    pltpu.sync_copy(zbuf, dsrc_sp.at[pl.ds(s * RPT, RPT)])
    pltpu.sync_copy(zbuf, ddst_sp.at[pl.ds(s * RPT, RPT)])
    plsc.subcore_barrier()

    def _body(i, _):
        pltpu.async_copy(ones, dsrc_sp.at[sidx.at[i]], sem_s, add=True)
        pltpu.async_copy(ones, ddst_sp.at[didx.at[i]], sem_d, add=True)

        @pl.when(i >= DEPTH)
        def _w():
            j = i - DEPTH
            pltpu.make_async_copy(ones, dsrc_sp.at[sidx.at[j]], sem_s).wait()
            pltpu.make_async_copy(ones, ddst_sp.at[didx.at[j]], sem_d).wait()
        return _
    lax.fori_loop(0, NCH1, _body, None)

    def _drain(i, _):
        j = NCH1 - DEPTH + i
        pltpu.make_async_copy(ones, dsrc_sp.at[sidx.at[j]], sem_s).wait()
        pltpu.make_async_copy(ones, ddst_sp.at[didx.at[j]], sem_d).wait()
        return _
    lax.fori_loop(0, DEPTH, _drain, None)
    plsc.subcore_barrier()

    pltpu.sync_copy(dsrc_sp.at[pl.ds(s * RPT, RPT)],
                    dsrc_out.at[c].at[pl.ds(s * RPT, RPT)])
    pltpu.sync_copy(ddst_sp.at[pl.ds(s * RPT, RPT)],
                    ddst_out.at[c].at[pl.ds(s * RPT, RPT)])


def _mm_body(x_ref, w_ref, dso_ref, dsi_ref, h_ref, rout_ref, rin_ref):
    do = dso_ref[0] + dso_ref[1]
    di = dsi_ref[0] + dsi_ref[1]
    ro = lax.rsqrt(jnp.maximum(do, 1.0))
    ri = lax.rsqrt(jnp.maximum(di, 1.0))
    rout_ref[...] = ro
    rin_ref[...] = ri
    h_ref[...] = jnp.dot(x_ref[...] * ro, w_ref[...],
                         preferred_element_type=jnp.float32)


def _matmul(x_pad, W1, dsrc_p, ddst_p):
    return pl.pallas_call(
        _mm_body,
        grid=(GR,),
        in_specs=[
            pl.BlockSpec((BR, F), lambda i: (i, 0)),
            pl.BlockSpec((F, F), lambda i: (0, 0)),
            pl.BlockSpec((NC, BR, 1), lambda i: (0, i, 0)),
            pl.BlockSpec((NC, BR, 1), lambda i: (0, i, 0)),
        ],
        out_specs=[
            pl.BlockSpec((BR, F), lambda i: (i, 0)),
            pl.BlockSpec((BR, 1), lambda i: (i, 0)),
            pl.BlockSpec((BR, 1), lambda i: (i, 0)),
        ],
        out_shape=[
            jax.ShapeDtypeStruct((NPAD, F), jnp.float32),
            jax.ShapeDtypeStruct((NPAD, 1), jnp.float32),
            jax.ShapeDtypeStruct((NPAD, 1), jnp.float32),
        ],
    )(x_pad, W1, dsrc_p, ddst_p)


@functools.partial(
    pl.kernel,
    out_type=(jax.ShapeDtypeStruct((NC, NPAD, F), jnp.float32),
              jax.ShapeDtypeStruct((NC, NPAD), jnp.float32)),
    mesh=_MESH,
    scratch_types=(
        pltpu.VMEM((PH0, CH), jnp.int32),
        pltpu.VMEM((PH0, CH), jnp.int32),
        pltpu.VMEM((CH, F), jnp.float32),
        pltpu.VMEM((CH, F), jnp.float32),
        pltpu.VMEM((CH,), jnp.float32),
        pltpu.VMEM((CH,), jnp.float32),
        pltpu.VMEM_SHARED((NPAD, F), jnp.float32),
        pltpu.VMEM_SHARED((NPAD,), jnp.float32),
        pltpu.VMEM_SHARED((NPAD,), jnp.float32),
        pltpu.SemaphoreType.DMA,
        pltpu.SemaphoreType.DMA,
        pltpu.SemaphoreType.DMA,
        pltpu.SemaphoreType.DMA,
        pltpu.SemaphoreType.DMA,
        pltpu.SemaphoreType.DMA,
        pltpu.SemaphoreType.DMA,
        pltpu.SemaphoreType.DMA,
    ),
)
def _propagate(h_hbm, src_hbm, dst_hbm, rin_hbm, agg_out, c_out,
               sidx, didx, rows0, rows1, rvals0, rvals1,
               agg_sp, c_sp, rin_sp, gsem0, gsem1, ssem0, ssem1,
               rgsem0, rgsem1, rssem0, rssem1):
    c = lax.axis_index("c")
    s = lax.axis_index("s")
    wid = s * NC + c

    def _zero(i, _):
        r = i // (F // 16)
        k = i % (F // 16)
        rows0[r, pl.ds(k * 16, 16)] = jnp.zeros((16,), jnp.float32)
        return _
    lax.fori_loop(0, CH * (F // 16), _zero, None)

    for k in range(RPT // CH):
        pltpu.sync_copy(rows0, agg_sp.at[pl.ds(s * RPT + k * CH, CH)])
    for k in range(RPT // F):
        pltpu.sync_copy(rows0.at[0], c_sp.at[pl.ds(s * RPT + k * F, F)])

    @pl.when(s == 0)
    def _stage_rin():
        pltpu.sync_copy(rin_hbm, rin_sp)
    plsc.subcore_barrier()

    rbufs = ((rows0, gsem0, ssem0), (rows1, gsem1, ssem1))
    cbufs = ((rvals0, rgsem0, rssem0), (rvals1, rgsem1, rssem1))

    def _start_gather(j, b):
        rows, gsem, _ = rbufs[b]
        pltpu.async_copy(h_hbm.at[sidx.at[j]], rows, gsem)

    def _wait_gather(j, b):
        rows, gsem, _ = rbufs[b]
        pltpu.make_async_copy(h_hbm.at[sidx.at[j]], rows, gsem).wait()

    def _start_scatter(j, b):
        rows, _, ssem = rbufs[b]
        pltpu.async_copy(rows, agg_sp.at[didx.at[j]], ssem, add=True)

    def _wait_scatter(j, b):
        rows, _, ssem = rbufs[b]
        pltpu.make_async_copy(rows, agg_sp.at[didx.at[j]], ssem).wait()

    def _start_rgather(j, b):
        rvals, rgsem, _ = cbufs[b]
        pltpu.async_copy(rin_sp.at[didx.at[j]], rvals, rgsem)

    def _wait_rgather(j, b):
        rvals, rgsem, _ = cbufs[b]
        pltpu.make_async_copy(rin_sp.at[didx.at[j]], rvals, rgsem).wait()

    def _start_cscatter(j, b):
        rvals, _, rssem = cbufs[b]
        pltpu.async_copy(rvals, c_sp.at[sidx.at[j]], rssem, add=True)

    def _wait_cscatter(j, b):
        rvals, _, rssem = cbufs[b]
        pltpu.make_async_copy(rvals, c_sp.at[sidx.at[j]], rssem).wait()

    def _phase(glo, cnt):
        pltpu.sync_copy(src_hbm.at[wid].at[pl.ds(glo, cnt)],
                        sidx.at[pl.ds(0, cnt)])
        pltpu.sync_copy(dst_hbm.at[wid].at[pl.ds(glo, cnt)],
                        didx.at[pl.ds(0, cnt)])

        _start_gather(0, 0)

        def _body(i, _):
            j = i * 2
            _wait_gather(j, 0)

            @pl.when(i > 0)
            def _():
                _wait_scatter(j - 1, 1)
            _start_gather(j + 1, 1)
            _start_scatter(j, 0)

            @pl.when(i > 0)
            def _():
                _wait_cscatter(j - 2, 0)
                _wait_cscatter(j - 1, 1)
            _start_rgather(j, 0)
            _start_rgather(j + 1, 1)

            _wait_gather(j + 1, 1)
            _wait_scatter(j, 0)
            _start_gather(j + 2, 0)
            _start_scatter(j + 1, 1)

            _wait_rgather(j, 0)
            _start_cscatter(j, 0)
            _wait_rgather(j + 1, 1)
            _start_cscatter(j + 1, 1)
            return _
        nb = (cnt - 1) // 2
        lax.fori_loop(0, nb, _body, None)

        if cnt % 2 == 0:
            _wait_gather(cnt - 2, 0)
            _wait_scatter(cnt - 3, 1)
            _start_gather(cnt - 1, 1)
            _start_scatter(cnt - 2, 0)
            _wait_gather(cnt - 1, 1)
            _wait_scatter(cnt - 2, 0)
            _start_scatter(cnt - 1, 1)
            _wait_scatter(cnt - 1, 1)
            _wait_cscatter(cnt - 4, 0)
            _wait_cscatter(cnt - 3, 1)
            _start_rgather(cnt - 2, 0)
            _start_rgather(cnt - 1, 1)
            _wait_rgather(cnt - 2, 0)
            _start_cscatter(cnt - 2, 0)
            _wait_rgather(cnt - 1, 1)
            _start_cscatter(cnt - 1, 1)
            _wait_cscatter(cnt - 2, 0)
            _wait_cscatter(cnt - 1, 1)
        else:
            _wait_gather(cnt - 1, 0)
            _wait_scatter(cnt - 2, 1)
            _start_scatter(cnt - 1, 0)
            _wait_scatter(cnt - 1, 0)
            _wait_cscatter(cnt - 3, 0)
            _wait_cscatter(cnt - 2, 1)
            _start_rgather(cnt - 1, 0)
            _wait_rgather(cnt - 1, 0)
            _start_cscatter(cnt - 1, 0)
            _wait_cscatter(cnt - 1, 0)

    _phase(0, PH0)
    _phase(PH0, NCHUNK - PH0)
    plsc.subcore_barrier()

    pltpu.sync_copy(agg_sp.at[pl.ds(s * RPT, RPT)],
                    agg_out.at[c].at[pl.ds(s * RPT, RPT)])
    pltpu.sync_copy(c_sp.at[pl.ds(s * RPT, RPT)],
                    c_out.at[c].at[pl.ds(s * RPT, RPT)])


def _fin_body(agg_ref, c_ref, rin_ref, rout_ref, b1_ref, w2_ref, b2_ref,
              out_ref, acc_ref):
    i = pl.program_id(0)
    a = agg_ref[0] + agg_ref[1]
    g = a * rin_ref[...] + b1_ref[...]
    h1 = jnp.maximum(g, 0.0)
    row = jax.lax.broadcasted_iota(jnp.int32, (BR, 1), 0) + i * BR
    w = jnp.where(row < N, (c_ref[0] + c_ref[1]) * rout_ref[...], 0.0)
    part = jnp.sum((h1 * w).reshape(BR // 8, 8, F), axis=0)

    @pl.when(i == 0)
    def _():
        acc_ref[...] = part

    @pl.when(i > 0)
    def _():
        acc_ref[...] = acc_ref[...] + part

    @pl.when(i == GR - 1)
    def _():
        sv = jnp.sum(acc_ref[...], axis=0, keepdims=True)
        out_ref[...] = (jnp.dot(sv, w2_ref[...],
                                preferred_element_type=jnp.float32)
                        * (1.0 / N) + b2_ref[...])


def _finalize(agg_p, c_p, r_in, r_out, b1, W2, b2):
    return pl.pallas_call(
        _fin_body,
        grid=(GR,),
        in_specs=[
            pl.BlockSpec((NC, BR, F), lambda i: (0, i, 0)),
            pl.BlockSpec((NC, BR, 1), lambda i: (0, i, 0)),
            pl.BlockSpec((BR, 1), lambda i: (i, 0)),
            pl.BlockSpec((BR, 1), lambda i: (i, 0)),
            pl.BlockSpec((1, F), lambda i: (0, 0)),
            pl.BlockSpec((F, C), lambda i: (0, 0)),
            pl.BlockSpec((1, C), lambda i: (0, 0)),
        ],
        out_specs=pl.BlockSpec((1, C), lambda i: (0, 0)),
        out_shape=jax.ShapeDtypeStruct((1, C), jnp.float32),
        scratch_shapes=[pltpu.VMEM((8, F), jnp.float32)],
    )(agg_p, c_p, r_in, r_out, b1, W2, b2)


def kernel(x, edge_index, W1, b1, W2, b2):
    src = edge_index[0].astype(jnp.int32)
    dst = edge_index[1].astype(jnp.int32)
    src32 = src.reshape(NW, NCH1, CH1)
    dst32 = dst.reshape(NW, NCH1, CH1)
    npad_e = EPW_PAD - EPW
    pad_rows = (jnp.arange(NW * npad_e, dtype=jnp.int32).reshape(NW, npad_e)
                % (NPAD - N)) + N
    srcp = jnp.concatenate(
        [src.reshape(NW, EPW), pad_rows], axis=1).reshape(NW, NCHUNK, CH)
    dstp = jnp.concatenate(
        [dst.reshape(NW, EPW), pad_rows], axis=1).reshape(NW, NCHUNK, CH)
    x_pad = jnp.zeros((NPAD, F), jnp.float32).at[:N].set(x)
    dsrc_p, ddst_p = _degrees(src32, dst32)
    h, r_out, r_in = _matmul(x_pad, W1,
                             dsrc_p.reshape(NC, NPAD, 1),
                             ddst_p.reshape(NC, NPAD, 1))
    agg_p, c_p = _propagate(h, srcp, dstp, r_in.reshape(NPAD))
    out = _finalize(agg_p, c_p.reshape(NC, NPAD, 1), r_in, r_out,
                    b1.reshape(1, F), W2, b2.reshape(1, C))
    return out
